# baseline (speedup 1.0000x reference)
import functools

import jax
import jax.numpy as jnp
from jax import lax
from jax.experimental import pallas as pl
from jax.experimental.pallas import tpu as pltpu
from jax.experimental.pallas import tpu_sc as plsc

_D = 128
_G = 1024
_NW = 32
_GPW = _G // _NW
_CH = 256
_NLANE = 16
_NVEC = _D // _NLANE


def _stv(st_ref, j):
    grp = j // _NLANE
    lane = j % _NLANE
    vec = st_ref[pl.ds(grp * _NLANE, _NLANE)]
    lanes = lax.iota(jnp.int32, _NLANE)
    return jnp.max(jnp.where(lanes == lane, vec, 0))


def _row_vecs(buf, r):
    return [buf[r, pl.ds(k * _NLANE, _NLANE)] for k in range(_NVEC)]


_NST = _G + _NLANE
_NSTG = _NST // _NLANE


def _make_seg_kernel(weighted: bool, n_src: int):
    mesh = plsc.VectorSubcoreMesh(core_axis_name="c", subcore_axis_name="s",
                                  num_cores=2, num_subcores=16)
    rpw = -(-n_src // 16)
    lpad = ((rpw + 16) + 7) // 8 * 8
    if weighted:
        out_type = [
            jax.ShapeDtypeStruct((_G, _D), jnp.float32),
            jax.ShapeDtypeStruct((_G, _NLANE), jnp.float32),
        ]
    else:
        out_type = [
            jax.ShapeDtypeStruct((_G, _D), jnp.float32),
            jax.ShapeDtypeStruct((_NST,), jnp.int32),
        ]
    scratch = [
        pltpu.VMEM((_CH, _D), jnp.float32),
        pltpu.VMEM((_CH, _D), jnp.float32),
        pltpu.VMEM((48,), jnp.int32),
        pltpu.VMEM((_GPW, _D), jnp.float32),
        pltpu.VMEM((_GPW, _NLANE), jnp.float32),
        pltpu.VMEM((_GPW, _D), jnp.float32),
        pltpu.VMEM((_D,), jnp.float32),
        pltpu.SemaphoreType.DMA,
        pltpu.SemaphoreType.DMA,
    ]
    if not weighted:
        scratch += [
            pltpu.VMEM((lpad + 16,), jnp.int32),
            pltpu.VMEM((_NST,), jnp.int32),
            pltpu.VMEM((_NST,), jnp.int32),
            pltpu.VMEM((_NST,), jnp.int32),
            pltpu.VMEM_SHARED((16 * _NST,), jnp.int32),
        ]

    def body(*refs):
        if weighted:
            (x_hbm, xr_hbm, att_hbm, st_hbm,
             v_hbm, s_hbm,
             buf0, buf1, st_v, vacc, sacc, xr_v, att_v, sem0, sem1) = refs
        else:
            (x_hbm, batch_hbm,
             v_hbm, st_out_hbm,
             buf0, buf1, st_v, vacc, sacc, xr_v, att_v, sem0, sem1,
             bbuf, st_loc, macc, mtmp, shared) = refs

        wid = lax.axis_index("s") * 2 + lax.axis_index("c")
        g0 = wid * _GPW
        lanes = lax.iota(jnp.int32, _NLANE)
        if weighted:
            pltpu.sync_copy(st_hbm.at[pl.ds(g0, 48)], st_v)
            pltpu.sync_copy(xr_hbm.at[pl.ds(g0, _GPW)], xr_v)
            pltpu.sync_copy(att_hbm, att_v)
            att_k = [att_v[pl.ds(k * _NLANE, _NLANE)] for k in range(_NVEC)]
        else:
            sid = lax.axis_index("s")
            cid = lax.axis_index("c")
            n32 = jnp.int32(n_src)
            row0 = sid * rpw
            limrows = jnp.maximum(jnp.minimum(rpw, n32 - row0), 0)
            sent = jnp.full((_NLANE,), n_src, jnp.int32)
            for j in range(_NSTG):
                st_loc[pl.ds(j * _NLANE, _NLANE)] = sent
            bbuf[pl.ds(0, _NLANE)] = jnp.full((_NLANE,), -1, jnp.int32)
            lo = jnp.where(sid == 0, 0, (row0 - 1) // 8 * 8).astype(jnp.int32)
            pltpu.sync_copy(batch_hbm.at[pl.ds(lo, lpad)],
                            bbuf.at[pl.ds(_NLANE, lpad)])
            off = _NLANE + (row0 - lo)
            nscan = (limrows + _NLANE - 1) // _NLANE

            def scan_body(i, _):
                p = off + i * _NLANE
                cur = bbuf[pl.ds(p, _NLANE)]
                prev = bbuf[pl.ds(p - 1, _NLANE)]
                valid = (i * _NLANE + lanes) < limrows
                tmask = jnp.logical_and(cur != prev, valid)
                rowvec = jnp.full((_NLANE,), row0 + i * _NLANE,
                                  jnp.int32) + lanes
                plsc.store_scatter(st_loc, [cur], rowvec, mask=tmask)
                return 0

            lax.fori_loop(0, nscan, scan_body, 0)
            pltpu.sync_copy(st_loc, shared.at[pl.ds(sid * _NST, _NST)])
            plsc.subcore_barrier()

            @pl.when(sid == 0)
            def _merge():
                pltpu.sync_copy(shared.at[pl.ds(0, _NST)], macc)

                def merge_body(w2, _):
                    pltpu.sync_copy(shared.at[pl.ds(w2 * _NST, _NST)], mtmp)
                    for j in range(_NSTG):
                        sl = pl.ds(j * _NLANE, _NLANE)
                        macc[sl] = jnp.minimum(macc[sl], mtmp[sl])
                    return 0

                lax.fori_loop(1, 16, merge_body, 0)
                carry = sent
                for j in range(_NSTG - 1, -1, -1):
                    sl = pl.ds(j * _NLANE, _NLANE)
                    vv = macc[sl]
                    u = lax.rev(vv, (0,))
                    cm = lax.rev(-plsc.cummax(-u), (0,))
                    outv = jnp.minimum(cm, carry)
                    macc[sl] = outv
                    carry = jnp.full((_NLANE,), outv[0], jnp.int32)
                pltpu.sync_copy(macc, shared.at[pl.ds(0, _NST)])

            @pl.when(jnp.logical_and(sid == 0, cid == 0))
            def _writeout():
                pltpu.sync_copy(macc, st_out_hbm)

            plsc.subcore_barrier()
            pltpu.sync_copy(shared.at[pl.ds(g0, 48)], st_v)

        zero = jnp.zeros((_NLANE,), jnp.float32)
        neg_inf = jnp.full((_NLANE,), -jnp.inf, jnp.float32)

        def zero_body(gl, _):
            for k in range(_NVEC):
                vacc[gl, pl.ds(k * _NLANE, _NLANE)] = zero
            sacc[gl, :] = zero
            return 0
        lax.fori_loop(0, _GPW, zero_body, 0)

        r0 = _stv(st_v, 0)
        r1 = _stv(st_v, _GPW)
        r0a = (r0 // 8) * 8
        nch = (r1 - r0a + _CH - 1) // _CH
        npair = (nch + 1) // 2

        clamp_hi = (n_src - _CH) // 8 * 8

        def dma(base, buf, sem):
            basec = jnp.minimum(base, clamp_hi)
            return pltpu.make_async_copy(x_hbm.at[pl.ds(basec, _CH)], buf, sem)

        def init_state():
            return (neg_inf, zero, tuple(zero for _ in range(_NVEC)))

        def process_chunk(buf, base, carry):
            pos, gl, m, s, v = carry
            lim = jnp.minimum(base + _CH, r1)
            base = jnp.minimum(base, clamp_hi)

            def while_cond(c):
                return c[0] < lim

            def while_body(c):
                pos, gl, m, s, v = c
                end_g = _stv(st_v, gl + 1)
                stop = jnp.minimum(end_g, lim)
                if weighted:
                    xr_k = [xr_v[gl, pl.ds(k * _NLANE, _NLANE)]
                            for k in range(_NVEC)]

                    def row_body(r, c2):
                        m, s, v = c2
                        row = _row_vecs(buf, r)
                        acc = [zero, zero]
                        for k in range(_NVEC):
                            t = row[k] + xr_k[k]
                            t = jnp.maximum(t, 0.01 * t)
                            acc[k % 2] = acc[k % 2] + t * att_k[k]
                        alpha = jnp.full((_NLANE,),
                                         jnp.sum(acc[0] + acc[1]))
                        mn = jnp.maximum(m, alpha)
                        corr = jnp.exp(m - mn)
                        w = jnp.exp(alpha - mn)
                        s2 = s * corr + w
                        v2 = tuple(v[k] * corr + w * row[k]
                                   for k in range(_NVEC))
                        return (mn, s2, v2)

                    m, s, v = lax.fori_loop(pos - base, stop - base,
                                            row_body, (m, s, v))
                else:
                    def row_body(r, c2):
                        m, s, v = c2
                        row = _row_vecs(buf, r)
                        return (m, s, tuple(v[k] + row[k]
                                            for k in range(_NVEC)))

                    m, s, v = lax.fori_loop(pos - base, stop - base,
                                            row_body, (m, s, v))
                for k in range(_NVEC):
                    vacc[gl, pl.ds(k * _NLANE, _NLANE)] = v[k]
                if weighted:
                    sacc[gl, :] = s
                finished = stop == end_g
                gl = gl + jnp.where(finished, 1, 0).astype(jnp.int32)
                m = jnp.where(finished, neg_inf, m)
                s = jnp.where(finished, zero, s)
                v = tuple(jnp.where(finished, zero, v[k])
                          for k in range(_NVEC))
                return (stop, gl, m, s, v)

            pos, gl, m, s, v = lax.while_loop(
                while_cond, while_body, (pos, gl, m, s, v))
            return (pos, gl, m, s, v)

        dma(r0a, buf0, sem0).start()

        def pair_body(p, carry):
            c = 2 * p
            base0 = r0a + c * _CH
            base1 = base0 + _CH
            dma(base0, buf0, sem0).wait()
            dma(base1, buf1, sem1).start()
            carry = process_chunk(buf0, base0, carry)
            dma(base1, buf1, sem1).wait()
            dma(base1 + _CH, buf0, sem0).start()
            carry = process_chunk(buf1, base1, carry)
            return carry

        m0, s0, v0 = init_state()
        carry = (r0, jnp.int32(0), m0, s0, v0)
        carry = lax.fori_loop(0, npair, pair_body, carry)
        dma(r0a, buf0, sem0).wait()

        pltpu.sync_copy(vacc, v_hbm.at[pl.ds(g0, _GPW)])
        if weighted:
            pltpu.sync_copy(sacc, s_hbm.at[pl.ds(g0, _GPW)])

    return pl.kernel(body, out_type=out_type, mesh=mesh,
                     scratch_types=scratch,
                     compiler_params=pltpu.CompilerParams(
                         needs_layout_passes=False),
                     name="sc_attn_pool" if weighted else "sc_seg_sum")


def _dotT(a, w):
    return lax.dot_general(a, w, (((1,), (1,)), ((), ())),
                           preferred_element_type=jnp.float32)


def _xl_body(x_ref, w_ref, b_ref, o_ref):
    o_ref[...] = lax.dot_general(
        x_ref[...], w_ref[...], (((1,), (0,)), ((), ())),
        preferred_element_type=jnp.float32) + b_ref[...]


def _init_body(seg_ref, lrw_ref, lrb_ref, out_ref, xr_ref):
    o = jnp.maximum(seg_ref[...], 0.0)
    out_ref[...] = o
    xr_ref[...] = _dotT(o, lrw_ref[...]) + lrb_ref[...]


def _gru_body(v_ref, s_ref, out_ref, wih_ref, whh_ref, bih_ref, bhh_ref,
              gatb_ref, lrw_ref, lrb_ref, onew_ref, xr_ref):
    s = s_ref[...][:, 0:1] + 1e-16
    h = v_ref[...] / s + gatb_ref[...]
    h = jnp.where(h > 0, h, jnp.exp(jnp.minimum(h, 0.0)) - 1.0)
    out = out_ref[...]
    gi = _dotT(h, wih_ref[...]) + bih_ref[...]
    gh = _dotT(out, whh_ref[...]) + bhh_ref[...]
    i_r, i_z, i_n = gi[:, :_D], gi[:, _D:2 * _D], gi[:, 2 * _D:]
    h_r, h_z, h_n = gh[:, :_D], gh[:, _D:2 * _D], gh[:, 2 * _D:]
    r = jax.nn.sigmoid(i_r + h_r)
    z = jax.nn.sigmoid(i_z + h_z)
    cand = jnp.tanh(i_n + r * h_n)
    onew = jnp.maximum((1.0 - z) * cand + z * out, 0.0)
    onew_ref[...] = onew
    xr_ref[...] = _dotT(onew, lrw_ref[...]) + lrb_ref[...]


def _fc_body(out_ref, w_ref, b_ref, y_ref):
    y_ref[...] = _dotT(out_ref[...], w_ref[...]) + b_ref[...]


def kernel(x, batch, lin_l_w, lin_l_b, lin_r_w, lin_r_b, att, gat_bias,
           w_ih, w_hh, b_ih, b_hh, fc_w, fc_b):
    n = x.shape[0]
    tile = 2048

    n2 = n // 2
    x2 = x[:2 * n2].reshape(n2, 2 * _D)
    w_bd = jnp.zeros((2 * _D, 2 * _D), jnp.float32)
    w_bd = w_bd.at[:_D, :_D].set(lin_l_w.T).at[_D:, _D:].set(lin_l_w.T)
    b2 = jnp.concatenate([lin_l_b, lin_l_b]).reshape(1, 2 * _D)
    num_tiles = (n2 + tile - 1) // tile
    x_l2 = pl.pallas_call(
        _xl_body,
        grid=(num_tiles,),
        in_specs=[
            pl.BlockSpec((tile, 2 * _D), lambda i: (i, 0)),
            pl.BlockSpec((2 * _D, 2 * _D), lambda i: (0, 0)),
            pl.BlockSpec((1, 2 * _D), lambda i: (0, 0)),
        ],
        out_specs=pl.BlockSpec((tile, 2 * _D), lambda i: (i, 0)),
        out_shape=jax.ShapeDtypeStruct((n2, 2 * _D), jnp.float32),
    )(x2, w_bd, b2)
    x_l = x_l2.reshape(2 * n2, _D)
    if 2 * n2 != n:
        tail = x[2 * n2:] @ lin_l_w.T + lin_l_b
        x_l = jnp.concatenate([x_l, tail], axis=0)

    seg_sum = _make_seg_kernel(False, n)
    attn_pool = _make_seg_kernel(True, n)

    rpw16 = -(-n // 16)
    batch_p = jnp.full((16 * rpw16 + 64,), _G, jnp.int32).at[:n].set(batch)
    seg0, starts_p = seg_sum(x, batch_p)

    out, xr = pl.pallas_call(
        _init_body,
        out_shape=[jax.ShapeDtypeStruct((_G, _D), jnp.float32),
                   jax.ShapeDtypeStruct((_G, _D), jnp.float32)],
    )(seg0, lin_r_w, lin_r_b.reshape(1, _D))

    for _ in range(3):
        v, s = attn_pool(x_l, xr, att, starts_p)
        out, xr = pl.pallas_call(
            _gru_body,
            out_shape=[jax.ShapeDtypeStruct((_G, _D), jnp.float32),
                       jax.ShapeDtypeStruct((_G, _D), jnp.float32)],
        )(v, s, out, w_ih, w_hh, b_ih.reshape(1, 3 * _D),
          b_hh.reshape(1, 3 * _D), gat_bias.reshape(1, _D),
          lin_r_w, lin_r_b.reshape(1, _D))

    return pl.pallas_call(
        _fc_body,
        out_shape=jax.ShapeDtypeStruct((_G, _D), jnp.float32),
    )(out, fc_w, fc_b.reshape(1, _D))


# speedup vs baseline: 2.2536x; 2.2536x over previous
import functools

import jax
import jax.numpy as jnp
from jax import lax
from jax.experimental import pallas as pl
from jax.experimental.pallas import tpu as pltpu
from jax.experimental.pallas import tpu_sc as plsc

_D = 128
_G = 1024
_NW = 32
_GPW = _G // _NW
_CH = 256
_NLANE = 16
_NVEC = _D // _NLANE


def _stv(st_ref, j):
    grp = j // _NLANE
    lane = j % _NLANE
    vec = st_ref[pl.ds(grp * _NLANE, _NLANE)]
    lanes = lax.iota(jnp.int32, _NLANE)
    return jnp.max(jnp.where(lanes == lane, vec, 0))


def _row_vecs(buf, r):
    return [buf[r, pl.ds(k * _NLANE, _NLANE)] for k in range(_NVEC)]


_NST = _G + _NLANE
_NSTG = _NST // _NLANE


def _make_seg_kernel(weighted: bool, n_src: int):
    mesh = plsc.VectorSubcoreMesh(core_axis_name="c", subcore_axis_name="s",
                                  num_cores=2, num_subcores=16)
    rpw = -(-n_src // 16)
    lpad = ((rpw + 16) + 7) // 8 * 8
    if weighted:
        out_type = [
            jax.ShapeDtypeStruct((_G, _D), jnp.float32),
            jax.ShapeDtypeStruct((_G, _NLANE), jnp.float32),
        ]
    else:
        out_type = [
            jax.ShapeDtypeStruct((_G, _D), jnp.float32),
            jax.ShapeDtypeStruct((_NST,), jnp.int32),
        ]
    scratch = [
        pltpu.VMEM((_CH, _D), jnp.float32),
        pltpu.VMEM((_CH, _D), jnp.float32),
        pltpu.VMEM((48,), jnp.int32),
        pltpu.VMEM((_GPW, _D), jnp.float32),
        pltpu.VMEM((_GPW, _NLANE), jnp.float32),
        pltpu.VMEM((_GPW, _D), jnp.float32),
        pltpu.VMEM((_D,), jnp.float32),
        pltpu.SemaphoreType.DMA,
        pltpu.SemaphoreType.DMA,
    ]
    if not weighted:
        scratch += [
            pltpu.VMEM((lpad + 16,), jnp.int32),
            pltpu.VMEM((_NST,), jnp.int32),
            pltpu.VMEM((_NST,), jnp.int32),
            pltpu.VMEM((_NST,), jnp.int32),
            pltpu.VMEM_SHARED((16 * _NST,), jnp.int32),
        ]

    def body(*refs):
        if weighted:
            (x_hbm, xr_hbm, att_hbm, st_hbm,
             v_hbm, s_hbm,
             buf0, buf1, st_v, vacc, sacc, xr_v, att_v, sem0, sem1) = refs
        else:
            (x_hbm, batch_hbm,
             v_hbm, st_out_hbm,
             buf0, buf1, st_v, vacc, sacc, xr_v, att_v, sem0, sem1,
             bbuf, st_loc, macc, mtmp, shared) = refs

        wid = lax.axis_index("s") * 2 + lax.axis_index("c")
        g0 = wid * _GPW
        lanes = lax.iota(jnp.int32, _NLANE)
        if weighted:
            pltpu.sync_copy(st_hbm.at[pl.ds(g0, 48)], st_v)
            pltpu.sync_copy(xr_hbm.at[pl.ds(g0, _GPW)], xr_v)
            pltpu.sync_copy(att_hbm, att_v)
            att_k = [att_v[pl.ds(k * _NLANE, _NLANE)] for k in range(_NVEC)]
        else:
            sid = lax.axis_index("s")
            cid = lax.axis_index("c")
            n32 = jnp.int32(n_src)
            row0 = sid * rpw
            limrows = jnp.maximum(jnp.minimum(rpw, n32 - row0), 0)
            sent = jnp.full((_NLANE,), n_src, jnp.int32)
            for j in range(_NSTG):
                st_loc[pl.ds(j * _NLANE, _NLANE)] = sent
            bbuf[pl.ds(0, _NLANE)] = jnp.full((_NLANE,), -1, jnp.int32)
            lo = jnp.where(sid == 0, 0, (row0 - 1) // 8 * 8).astype(jnp.int32)
            pltpu.sync_copy(batch_hbm.at[pl.ds(lo, lpad)],
                            bbuf.at[pl.ds(_NLANE, lpad)])
            off = _NLANE + (row0 - lo)
            nscan = (limrows + _NLANE - 1) // _NLANE

            def scan_body(i, _):
                p = off + i * _NLANE
                cur = bbuf[pl.ds(p, _NLANE)]
                prev = bbuf[pl.ds(p - 1, _NLANE)]
                valid = (i * _NLANE + lanes) < limrows
                tmask = jnp.logical_and(cur != prev, valid)
                rowvec = jnp.full((_NLANE,), row0 + i * _NLANE,
                                  jnp.int32) + lanes
                plsc.store_scatter(st_loc, [cur], rowvec, mask=tmask)
                return 0

            lax.fori_loop(0, nscan, scan_body, 0)
            pltpu.sync_copy(st_loc, shared.at[pl.ds(sid * _NST, _NST)])
            plsc.subcore_barrier()

            @pl.when(sid == 0)
            def _merge():
                pltpu.sync_copy(shared.at[pl.ds(0, _NST)], macc)

                def merge_body(w2, _):
                    pltpu.sync_copy(shared.at[pl.ds(w2 * _NST, _NST)], mtmp)
                    for j in range(_NSTG):
                        sl = pl.ds(j * _NLANE, _NLANE)
                        macc[sl] = jnp.minimum(macc[sl], mtmp[sl])
                    return 0

                lax.fori_loop(1, 16, merge_body, 0)
                carry = sent
                for j in range(_NSTG - 1, -1, -1):
                    sl = pl.ds(j * _NLANE, _NLANE)
                    vv = macc[sl]
                    u = lax.rev(vv, (0,))
                    cm = lax.rev(-plsc.cummax(-u), (0,))
                    outv = jnp.minimum(cm, carry)
                    macc[sl] = outv
                    carry = jnp.full((_NLANE,), outv[0], jnp.int32)
                pltpu.sync_copy(macc, shared.at[pl.ds(0, _NST)])

            @pl.when(jnp.logical_and(sid == 0, cid == 0))
            def _writeout():
                pltpu.sync_copy(macc, st_out_hbm)

            plsc.subcore_barrier()
            pltpu.sync_copy(shared.at[pl.ds(g0, 48)], st_v)

        zero = jnp.zeros((_NLANE,), jnp.float32)
        neg_inf = jnp.full((_NLANE,), -jnp.inf, jnp.float32)

        def zero_body(gl, _):
            for k in range(_NVEC):
                vacc[gl, pl.ds(k * _NLANE, _NLANE)] = zero
            sacc[gl, :] = zero
            return 0
        lax.fori_loop(0, _GPW, zero_body, 0)

        r0 = _stv(st_v, 0)
        r1 = _stv(st_v, _GPW)
        r0a = (r0 // 8) * 8
        nch = (r1 - r0a + _CH - 1) // _CH
        npair = (nch + 1) // 2

        clamp_hi = (n_src - _CH) // 8 * 8

        def dma(base, buf, sem):
            basec = jnp.minimum(base, clamp_hi)
            return pltpu.make_async_copy(x_hbm.at[pl.ds(basec, _CH)], buf, sem)

        def init_state():
            return (neg_inf, zero, tuple(zero for _ in range(_NVEC)))

        def process_chunk(buf, base, carry):
            pos, gl, m, s, v = carry
            lim = jnp.minimum(base + _CH, r1)
            base = jnp.minimum(base, clamp_hi)

            def while_cond(c):
                return c[0] < lim

            def while_body(c):
                pos, gl, m, s, v = c
                end_g = _stv(st_v, gl + 1)
                stop = jnp.minimum(end_g, lim)
                if weighted:
                    xr_k = [xr_v[gl, pl.ds(k * _NLANE, _NLANE)]
                            for k in range(_NVEC)]

                    def row_body(r, c2):
                        m, s, v = c2
                        row = _row_vecs(buf, r)
                        a = zero
                        for k in range(_NVEC):
                            t = row[k] + xr_k[k]
                            t = jnp.maximum(t, 0.01 * t)
                            a = a + t * att_k[k]
                        alpha = jnp.full((_NLANE,), jnp.sum(a))
                        mn = jnp.maximum(m, alpha)
                        corr = jnp.exp(m - mn)
                        w = jnp.exp(alpha - mn)
                        s2 = s * corr + w
                        v2 = tuple(v[k] * corr + w * row[k]
                                   for k in range(_NVEC))
                        return (mn, s2, v2)

                    m, s, v = lax.fori_loop(pos - base, stop - base,
                                            row_body, (m, s, v))
                else:
                    def row_body(r, c2):
                        m, s, v = c2
                        row = _row_vecs(buf, r)
                        return (m, s, tuple(v[k] + row[k]
                                            for k in range(_NVEC)))

                    m, s, v = lax.fori_loop(pos - base, stop - base,
                                            row_body, (m, s, v))
                for k in range(_NVEC):
                    vacc[gl, pl.ds(k * _NLANE, _NLANE)] = v[k]
                if weighted:
                    sacc[gl, :] = s
                finished = stop == end_g
                gl = gl + jnp.where(finished, 1, 0).astype(jnp.int32)
                m = jnp.where(finished, neg_inf, m)
                s = jnp.where(finished, zero, s)
                v = tuple(jnp.where(finished, zero, v[k])
                          for k in range(_NVEC))
                return (stop, gl, m, s, v)

            pos, gl, m, s, v = lax.while_loop(
                while_cond, while_body, (pos, gl, m, s, v))
            return (pos, gl, m, s, v)

        dma(r0a, buf0, sem0).start()

        def pair_body(p, carry):
            c = 2 * p
            base0 = r0a + c * _CH
            base1 = base0 + _CH
            dma(base0, buf0, sem0).wait()
            dma(base1, buf1, sem1).start()
            carry = process_chunk(buf0, base0, carry)
            dma(base1, buf1, sem1).wait()
            dma(base1 + _CH, buf0, sem0).start()
            carry = process_chunk(buf1, base1, carry)
            return carry

        m0, s0, v0 = init_state()
        carry = (r0, jnp.int32(0), m0, s0, v0)
        carry = lax.fori_loop(0, npair, pair_body, carry)
        dma(r0a, buf0, sem0).wait()

        pltpu.sync_copy(vacc, v_hbm.at[pl.ds(g0, _GPW)])
        if weighted:
            pltpu.sync_copy(sacc, s_hbm.at[pl.ds(g0, _GPW)])

    return pl.kernel(body, out_type=out_type, mesh=mesh,
                     scratch_types=scratch,
                     compiler_params=pltpu.CompilerParams(
                         needs_layout_passes=False),
                     name="sc_attn_pool" if weighted else "sc_seg_sum")


def _dotT(a, w):
    return lax.dot_general(a, w, (((1,), (1,)), ((), ())),
                           preferred_element_type=jnp.float32)


def _xl_body(x_ref, w_ref, b_ref, o_ref):
    o_ref[...] = _dotT(x_ref[...], w_ref[...]) + b_ref[...]


def _init_body(seg_ref, lrw_ref, lrb_ref, out_ref, xr_ref):
    o = jnp.maximum(seg_ref[...], 0.0)
    out_ref[...] = o
    xr_ref[...] = _dotT(o, lrw_ref[...]) + lrb_ref[...]


def _gru_body(v_ref, s_ref, out_ref, wih_ref, whh_ref, bih_ref, bhh_ref,
              gatb_ref, lrw_ref, lrb_ref, onew_ref, xr_ref):
    s = s_ref[...][:, 0:1] + 1e-16
    h = v_ref[...] / s + gatb_ref[...]
    h = jnp.where(h > 0, h, jnp.exp(jnp.minimum(h, 0.0)) - 1.0)
    out = out_ref[...]
    gi = _dotT(h, wih_ref[...]) + bih_ref[...]
    gh = _dotT(out, whh_ref[...]) + bhh_ref[...]
    i_r, i_z, i_n = gi[:, :_D], gi[:, _D:2 * _D], gi[:, 2 * _D:]
    h_r, h_z, h_n = gh[:, :_D], gh[:, _D:2 * _D], gh[:, 2 * _D:]
    r = jax.nn.sigmoid(i_r + h_r)
    z = jax.nn.sigmoid(i_z + h_z)
    cand = jnp.tanh(i_n + r * h_n)
    onew = jnp.maximum((1.0 - z) * cand + z * out, 0.0)
    onew_ref[...] = onew
    xr_ref[...] = _dotT(onew, lrw_ref[...]) + lrb_ref[...]


def _fc_body(out_ref, w_ref, b_ref, y_ref):
    y_ref[...] = _dotT(out_ref[...], w_ref[...]) + b_ref[...]


def kernel(x, batch, lin_l_w, lin_l_b, lin_r_w, lin_r_b, att, gat_bias,
           w_ih, w_hh, b_ih, b_hh, fc_w, fc_b):
    n = x.shape[0]
    tile = 2048

    num_tiles = (n + tile - 1) // tile
    x_l = pl.pallas_call(
        _xl_body,
        grid=(num_tiles,),
        in_specs=[
            pl.BlockSpec((tile, _D), lambda i: (i, 0)),
            pl.BlockSpec((_D, _D), lambda i: (0, 0)),
            pl.BlockSpec((1, _D), lambda i: (0, 0)),
        ],
        out_specs=pl.BlockSpec((tile, _D), lambda i: (i, 0)),
        out_shape=jax.ShapeDtypeStruct((n, _D), jnp.float32),
    )(x, lin_l_w, lin_l_b.reshape(1, _D))

    seg_sum = _make_seg_kernel(False, n)
    attn_pool = _make_seg_kernel(True, n)

    rpw16 = -(-n // 16)
    batch_p = jnp.full((16 * rpw16 + 64,), _G, jnp.int32).at[:n].set(batch)
    seg0, starts_p = seg_sum(x, batch_p)

    out, xr = pl.pallas_call(
        _init_body,
        out_shape=[jax.ShapeDtypeStruct((_G, _D), jnp.float32),
                   jax.ShapeDtypeStruct((_G, _D), jnp.float32)],
    )(seg0, lin_r_w, lin_r_b.reshape(1, _D))

    for _ in range(3):
        v, s = attn_pool(x_l, xr, att, starts_p)
        out, xr = pl.pallas_call(
            _gru_body,
            out_shape=[jax.ShapeDtypeStruct((_G, _D), jnp.float32),
                       jax.ShapeDtypeStruct((_G, _D), jnp.float32)],
        )(v, s, out, w_ih, w_hh, b_ih.reshape(1, 3 * _D),
          b_hh.reshape(1, 3 * _D), gat_bias.reshape(1, _D),
          lin_r_w, lin_r_b.reshape(1, _D))

    return pl.pallas_call(
        _fc_body,
        out_shape=jax.ShapeDtypeStruct((_G, _D), jnp.float32),
    )(out, fc_w, fc_b.reshape(1, _D))


# speedup vs baseline: 2.2878x; 1.0152x over previous
import functools

import jax
import jax.numpy as jnp
from jax import lax
from jax.experimental import pallas as pl
from jax.experimental.pallas import tpu as pltpu
from jax.experimental.pallas import tpu_sc as plsc

_D = 128
_G = 1024
_NW = 32
_GPW = _G // _NW
_CH = 256
_NLANE = 16
_NVEC = _D // _NLANE


def _stv(st_ref, j):
    grp = j // _NLANE
    lane = j % _NLANE
    vec = st_ref[pl.ds(grp * _NLANE, _NLANE)]
    lanes = lax.iota(jnp.int32, _NLANE)
    return jnp.max(jnp.where(lanes == lane, vec, 0))


def _row_vecs(buf, r):
    return [buf[r, pl.ds(k * _NLANE, _NLANE)] for k in range(_NVEC)]


_NST = _G + _NLANE
_NSTG = _NST // _NLANE


def _make_seg_kernel(weighted: bool, n_src: int):
    mesh = plsc.VectorSubcoreMesh(core_axis_name="c", subcore_axis_name="s",
                                  num_cores=2, num_subcores=16)
    rpw = -(-n_src // 16)
    lpad = ((rpw + 16) + 7) // 8 * 8
    if weighted:
        out_type = [
            jax.ShapeDtypeStruct((_G, _D), jnp.float32),
            jax.ShapeDtypeStruct((_G, _NLANE), jnp.float32),
        ]
    else:
        out_type = [
            jax.ShapeDtypeStruct((_G, _D), jnp.float32),
            jax.ShapeDtypeStruct((_NST,), jnp.int32),
        ]
    scratch = [
        pltpu.VMEM((_CH, _D), jnp.float32),
        pltpu.VMEM((_CH, _D), jnp.float32),
        pltpu.VMEM((48,), jnp.int32),
        pltpu.VMEM((_GPW, _D), jnp.float32),
        pltpu.VMEM((_GPW, _NLANE), jnp.float32),
        pltpu.VMEM((_GPW, _D), jnp.float32),
        pltpu.VMEM((_D,), jnp.float32),
        pltpu.SemaphoreType.DMA,
        pltpu.SemaphoreType.DMA,
    ]
    if not weighted:
        scratch += [
            pltpu.VMEM((lpad + 16,), jnp.int32),
            pltpu.VMEM((_NST,), jnp.int32),
            pltpu.VMEM((_NST,), jnp.int32),
            pltpu.VMEM((_NST,), jnp.int32),
            pltpu.VMEM_SHARED((16 * _NST,), jnp.int32),
        ]

    def body(*refs):
        if weighted:
            (x_hbm, xr_hbm, att_hbm, st_hbm,
             v_hbm, s_hbm,
             buf0, buf1, st_v, vacc, sacc, xr_v, att_v, sem0, sem1) = refs
        else:
            (x_hbm, batch_hbm,
             v_hbm, st_out_hbm,
             buf0, buf1, st_v, vacc, sacc, xr_v, att_v, sem0, sem1,
             bbuf, st_loc, macc, mtmp, shared) = refs

        wid = lax.axis_index("s") * 2 + lax.axis_index("c")
        g0 = wid * _GPW
        lanes = lax.iota(jnp.int32, _NLANE)
        if weighted:
            pltpu.sync_copy(st_hbm.at[pl.ds(g0, 48)], st_v)
            pltpu.sync_copy(xr_hbm.at[pl.ds(g0, _GPW)], xr_v)
            pltpu.sync_copy(att_hbm, att_v)
            att_k = [att_v[pl.ds(k * _NLANE, _NLANE)] for k in range(_NVEC)]
        else:
            sid = lax.axis_index("s")
            cid = lax.axis_index("c")
            n32 = jnp.int32(n_src)
            row0 = sid * rpw
            limrows = jnp.maximum(jnp.minimum(rpw, n32 - row0), 0)
            sent = jnp.full((_NLANE,), n_src, jnp.int32)
            for j in range(_NSTG):
                st_loc[pl.ds(j * _NLANE, _NLANE)] = sent
            bbuf[pl.ds(0, _NLANE)] = jnp.full((_NLANE,), -1, jnp.int32)
            lo = jnp.where(sid == 0, 0, (row0 - 1) // 8 * 8).astype(jnp.int32)
            pltpu.sync_copy(batch_hbm.at[pl.ds(lo, lpad)],
                            bbuf.at[pl.ds(_NLANE, lpad)])
            off = _NLANE + (row0 - lo)
            nscan = (limrows + _NLANE - 1) // _NLANE

            def scan_body(i, _):
                p = off + i * _NLANE
                cur = bbuf[pl.ds(p, _NLANE)]
                prev = bbuf[pl.ds(p - 1, _NLANE)]
                valid = (i * _NLANE + lanes) < limrows
                tmask = jnp.logical_and(cur != prev, valid)
                rowvec = jnp.full((_NLANE,), row0 + i * _NLANE,
                                  jnp.int32) + lanes
                plsc.store_scatter(st_loc, [cur], rowvec, mask=tmask)
                return 0

            lax.fori_loop(0, nscan, scan_body, 0)
            pltpu.sync_copy(st_loc, shared.at[pl.ds(sid * _NST, _NST)])
            plsc.subcore_barrier()

            @pl.when(sid == 0)
            def _merge():
                pltpu.sync_copy(shared.at[pl.ds(0, _NST)], macc)

                def merge_body(w2, _):
                    pltpu.sync_copy(shared.at[pl.ds(w2 * _NST, _NST)], mtmp)
                    for j in range(_NSTG):
                        sl = pl.ds(j * _NLANE, _NLANE)
                        macc[sl] = jnp.minimum(macc[sl], mtmp[sl])
                    return 0

                lax.fori_loop(1, 16, merge_body, 0)
                carry = sent
                for j in range(_NSTG - 1, -1, -1):
                    sl = pl.ds(j * _NLANE, _NLANE)
                    vv = macc[sl]
                    u = lax.rev(vv, (0,))
                    cm = lax.rev(-plsc.cummax(-u), (0,))
                    outv = jnp.minimum(cm, carry)
                    macc[sl] = outv
                    carry = jnp.full((_NLANE,), outv[0], jnp.int32)
                pltpu.sync_copy(macc, shared.at[pl.ds(0, _NST)])

            @pl.when(jnp.logical_and(sid == 0, cid == 0))
            def _writeout():
                pltpu.sync_copy(macc, st_out_hbm)

            plsc.subcore_barrier()
            pltpu.sync_copy(shared.at[pl.ds(g0, 48)], st_v)

        zero = jnp.zeros((_NLANE,), jnp.float32)
        neg_inf = jnp.full((_NLANE,), -jnp.inf, jnp.float32)

        def zero_body(gl, _):
            for k in range(_NVEC):
                vacc[gl, pl.ds(k * _NLANE, _NLANE)] = zero
            sacc[gl, :] = zero
            return 0
        lax.fori_loop(0, _GPW, zero_body, 0)

        r0 = _stv(st_v, 0)
        r1 = _stv(st_v, _GPW)
        r0a = (r0 // 8) * 8
        nch = (r1 - r0a + _CH - 1) // _CH
        npair = (nch + 1) // 2

        clamp_hi = (n_src - _CH) // 8 * 8

        def dma(base, buf, sem):
            basec = jnp.minimum(base, clamp_hi)
            return pltpu.make_async_copy(x_hbm.at[pl.ds(basec, _CH)], buf, sem)

        def init_state():
            return (neg_inf, zero, tuple(zero for _ in range(_NVEC)))

        def process_chunk(buf, base, carry):
            pos, gl, m, s, v = carry
            lim = jnp.minimum(base + _CH, r1)
            base = jnp.minimum(base, clamp_hi)

            def while_cond(c):
                return c[0] < lim

            def while_body(c):
                pos, gl, m, s, v = c
                end_g = _stv(st_v, gl + 1)
                stop = jnp.minimum(end_g, lim)
                if weighted:
                    xr_k = [xr_v[gl, pl.ds(k * _NLANE, _NLANE)]
                            for k in range(_NVEC)]

                    def row_body(r, c2):
                        m, s, v = c2
                        row = _row_vecs(buf, r)
                        a = zero
                        for k in range(_NVEC):
                            t = row[k] + xr_k[k]
                            t = jnp.maximum(t, 0.01 * t)
                            a = a + t * att_k[k]
                        alpha = jnp.full((_NLANE,), jnp.sum(a))
                        mn = jnp.maximum(m, alpha)
                        corr = jnp.exp(m - mn)
                        w = jnp.exp(alpha - mn)
                        s2 = s * corr + w
                        v2 = tuple(v[k] * corr + w * row[k]
                                   for k in range(_NVEC))
                        return (mn, s2, v2)

                    m, s, v = lax.fori_loop(pos - base, stop - base,
                                            row_body, (m, s, v))
                else:
                    def row_body(r, c2):
                        m, s, v = c2
                        row = _row_vecs(buf, r)
                        return (m, s, tuple(v[k] + row[k]
                                            for k in range(_NVEC)))

                    m, s, v = lax.fori_loop(pos - base, stop - base,
                                            row_body, (m, s, v))
                for k in range(_NVEC):
                    vacc[gl, pl.ds(k * _NLANE, _NLANE)] = v[k]
                if weighted:
                    sacc[gl, :] = s
                finished = stop == end_g
                gl = gl + jnp.where(finished, 1, 0).astype(jnp.int32)
                m = jnp.where(finished, neg_inf, m)
                s = jnp.where(finished, zero, s)
                v = tuple(jnp.where(finished, zero, v[k])
                          for k in range(_NVEC))
                return (stop, gl, m, s, v)

            pos, gl, m, s, v = lax.while_loop(
                while_cond, while_body, (pos, gl, m, s, v))
            return (pos, gl, m, s, v)

        dma(r0a, buf0, sem0).start()

        def pair_body(p, carry):
            c = 2 * p
            base0 = r0a + c * _CH
            base1 = base0 + _CH
            dma(base0, buf0, sem0).wait()
            dma(base1, buf1, sem1).start()
            carry = process_chunk(buf0, base0, carry)
            dma(base1, buf1, sem1).wait()
            dma(base1 + _CH, buf0, sem0).start()
            carry = process_chunk(buf1, base1, carry)
            return carry

        m0, s0, v0 = init_state()
        carry = (r0, jnp.int32(0), m0, s0, v0)
        carry = lax.fori_loop(0, npair, pair_body, carry)
        dma(r0a, buf0, sem0).wait()

        pltpu.sync_copy(vacc, v_hbm.at[pl.ds(g0, _GPW)])
        if weighted:
            pltpu.sync_copy(sacc, s_hbm.at[pl.ds(g0, _GPW)])

    return pl.kernel(body, out_type=out_type, mesh=mesh,
                     scratch_types=scratch,
                     compiler_params=pltpu.CompilerParams(
                         needs_layout_passes=False),
                     name="sc_attn_pool" if weighted else "sc_seg_sum")


def _dotT(a, w):
    return lax.dot_general(a, w, (((1,), (1,)), ((), ())),
                           preferred_element_type=jnp.float32)


def _xl_body(x_ref, w_ref, b_ref, o_ref):
    o_ref[...] = _dotT(x_ref[...], w_ref[...]) + b_ref[...]


def _init_body(seg_ref, lrw_ref, lrb_ref, out_ref, xr_ref):
    o = jnp.maximum(seg_ref[...], 0.0)
    out_ref[...] = o
    xr_ref[...] = _dotT(o, lrw_ref[...]) + lrb_ref[...]


def _gru_body(v_ref, s_ref, out_ref, wih_ref, whh_ref, bih_ref, bhh_ref,
              gatb_ref, lrw_ref, lrb_ref, fcw_ref, fcb_ref,
              onew_ref, xr_ref, y_ref):
    s = s_ref[...][:, 0:1] + 1e-16
    h = v_ref[...] / s + gatb_ref[...]
    h = jnp.where(h > 0, h, jnp.exp(jnp.minimum(h, 0.0)) - 1.0)
    out = out_ref[...]
    gi = _dotT(h, wih_ref[...]) + bih_ref[...]
    gh = _dotT(out, whh_ref[...]) + bhh_ref[...]
    i_r, i_z, i_n = gi[:, :_D], gi[:, _D:2 * _D], gi[:, 2 * _D:]
    h_r, h_z, h_n = gh[:, :_D], gh[:, _D:2 * _D], gh[:, 2 * _D:]
    r = jax.nn.sigmoid(i_r + h_r)
    z = jax.nn.sigmoid(i_z + h_z)
    cand = jnp.tanh(i_n + r * h_n)
    onew = jnp.maximum((1.0 - z) * cand + z * out, 0.0)
    onew_ref[...] = onew
    xr_ref[...] = _dotT(onew, lrw_ref[...]) + lrb_ref[...]
    y_ref[...] = _dotT(onew, fcw_ref[...]) + fcb_ref[...]


def _fc_body(out_ref, w_ref, b_ref, y_ref):
    y_ref[...] = _dotT(out_ref[...], w_ref[...]) + b_ref[...]


def kernel(x, batch, lin_l_w, lin_l_b, lin_r_w, lin_r_b, att, gat_bias,
           w_ih, w_hh, b_ih, b_hh, fc_w, fc_b):
    n = x.shape[0]
    tile = 4096

    num_tiles = (n + tile - 1) // tile
    x_l = pl.pallas_call(
        _xl_body,
        grid=(num_tiles,),
        in_specs=[
            pl.BlockSpec((tile, _D), lambda i: (i, 0)),
            pl.BlockSpec((_D, _D), lambda i: (0, 0)),
            pl.BlockSpec((1, _D), lambda i: (0, 0)),
        ],
        out_specs=pl.BlockSpec((tile, _D), lambda i: (i, 0)),
        out_shape=jax.ShapeDtypeStruct((n, _D), jnp.float32),
    )(x, lin_l_w, lin_l_b.reshape(1, _D))

    seg_sum = _make_seg_kernel(False, n)
    attn_pool = _make_seg_kernel(True, n)

    rpw16 = -(-n // 16)
    batch_p = jnp.full((16 * rpw16 + 64,), _G, jnp.int32).at[:n].set(batch)
    seg0, starts_p = seg_sum(x, batch_p)

    out, xr = pl.pallas_call(
        _init_body,
        out_shape=[jax.ShapeDtypeStruct((_G, _D), jnp.float32),
                   jax.ShapeDtypeStruct((_G, _D), jnp.float32)],
    )(seg0, lin_r_w, lin_r_b.reshape(1, _D))

    y = None
    for _ in range(3):
        v, s = attn_pool(x_l, xr, att, starts_p)
        out, xr, y = pl.pallas_call(
            _gru_body,
            out_shape=[jax.ShapeDtypeStruct((_G, _D), jnp.float32),
                       jax.ShapeDtypeStruct((_G, _D), jnp.float32),
                       jax.ShapeDtypeStruct((_G, _D), jnp.float32)],
        )(v, s, out, w_ih, w_hh, b_ih.reshape(1, 3 * _D),
          b_hh.reshape(1, 3 * _D), gat_bias.reshape(1, _D),
          lin_r_w, lin_r_b.reshape(1, _D), fc_w, fc_b.reshape(1, _D))

    return y
